# baseline (speedup 1.0000x reference)
import functools
import jax
import jax.numpy as jnp
from jax import lax
from jax.experimental import pallas as pl
from jax.experimental.pallas import tpu as pltpu
from jax.experimental.pallas import tpu_sc as plsc

B = 32
NPTS = 1024
N = B * NPTS
K = 20
EMB = 1024
OUT = 7
EPS = 1e-5
LANES = 128
BIG_F = 1e9

def _topk_kernel(pos_ref, post_ref, wa_ref, ba_ref, wc_ref, out_idx_ref,
                 a_ref, c_ref, dist_ref):
    b = pl.program_id(0)
    p = pos_ref[0]
    pt = post_ref[0]

    sq_r = jnp.sum(p * p, axis=1, keepdims=True)
    sq_c = jnp.sum(pt * pt, axis=0, keepdims=True)
    pp = jax.lax.dot(p, pt)
    dist_ref[...] = sq_r + sq_c - 2.0 * pp

    a_ref[0] = jax.lax.dot(p, wa_ref[...],
                           precision=jax.lax.Precision.HIGHEST) + ba_ref[...]
    c_ref[0] = jax.lax.dot(p, wc_ref[...],
                           precision=jax.lax.Precision.HIGHEST)

    iota = lax.broadcasted_iota(jnp.int32, (NPTS, NPTS), 1).astype(jnp.float32)
    lane = lax.broadcasted_iota(jnp.int32, (NPTS, LANES), 1)

    def body(k, carry):
        m, acc = carry
        d = dist_ref[...]
        loc = jnp.where(d == m, iota, jnp.float32(BIG_F))
        idx = jnp.min(loc, axis=1, keepdims=True)
        oh = iota == idx
        d2 = jnp.where(oh, jnp.inf, d)
        dist_ref[...] = d2
        m2 = jnp.min(d2, axis=1, keepdims=True)
        return m2, jnp.where(lane == k, idx, acc)

    m0 = jnp.min(dist_ref[...], axis=1, keepdims=True)
    acc0 = jnp.zeros((NPTS, LANES), jnp.float32)
    _, acc = lax.fori_loop(0, K, body, (m0, acc0))
    out_idx_ref[0] = acc.astype(jnp.int32) + b * NPTS


_CHUNK = 128
_SUPER = 512
_NC = 2
_NS = 16
_NW = _NC * _NS
_BPW = (K * N) // _NW
_NSUP = _BPW // _SUPER


def _sc_gather(table_hbm, idx_hbm, out_hbm, idx_v, stage_v, gsem):
    wid = lax.axis_index("s") * _NC + lax.axis_index("c")
    base = wid * _BPW
    pltpu.sync_copy(idx_hbm.at[pl.ds(base, _BPW)], idx_v)

    def body(g, carry):
        for c in range(_SUPER // _CHUNK):
            off = g * _SUPER + c * _CHUNK
            pltpu.async_copy(
                table_hbm.at[idx_v.at[pl.ds(off, _CHUNK)]],
                stage_v.at[pl.ds(c * _CHUNK, _CHUNK)], gsem)
        for c in range(_SUPER // _CHUNK):
            pltpu.make_async_copy(
                table_hbm.at[idx_v.at[pl.ds(g * _SUPER + c * _CHUNK, _CHUNK)]],
                stage_v.at[pl.ds(c * _CHUNK, _CHUNK)], gsem).wait()
        pltpu.sync_copy(stage_v, out_hbm.at[pl.ds(base + g * _SUPER, _SUPER)])
        return carry

    lax.fori_loop(0, _NSUP, body, 0)


def _edge_kernel(neigh_ref, a_ref, w2_ref, b2_ref, w3_ref, b3_ref,
                 w4a_ref, w4b_ref, b4_ref, out_ref):
    a = a_ref[0]
    x1 = jnp.zeros((NPTS, 128), jnp.float32)
    for k in range(K):
        h1 = jnp.maximum(a + neigh_ref[k, 0, :, :64], 0.0)
        h2 = jnp.maximum(jax.lax.dot(h1, w2_ref[...]) + b2_ref[...], 0.0)
        x1 = jnp.maximum(x1, h2)
    x = jnp.maximum(jax.lax.dot(x1, w3_ref[...]) + b3_ref[...], 0.0)
    gmax = jnp.max(x, axis=0, keepdims=True)
    gmean = jnp.sum(x, axis=0, keepdims=True) * (1.0 / NPTS)
    o = (jax.lax.dot(gmax, w4a_ref[...]) + jax.lax.dot(gmean, w4b_ref[...])
         + b4_ref[...])
    out_ref[0] = o


def kernel(pos, batch, W1, b1, g1, be1, W2, b2, g2, be2, W3, b3, g3, be3,
           W4, b4):
    del batch
    f32 = jnp.float32
    s1 = g1 / jnp.sqrt(1.0 + EPS)
    s2 = g2 / jnp.sqrt(1.0 + EPS)
    s3 = g3 / jnp.sqrt(1.0 + EPS)

    wa = (W1[:3] - W1[3:]) * s1[None, :]
    ba = (b1 * s1 + be1).reshape(1, 64)
    wc = jnp.zeros((3, LANES), f32).at[:, :64].set(W1[3:] * s1[None, :])
    w2 = W2 * s2[None, :]
    b2e = (b2 * s2 + be2).reshape(1, 128)
    w3 = W3 * s3[None, :]
    b3e = (b3 * s3 + be3).reshape(1, EMB)
    w4a = jnp.zeros((EMB, LANES), f32).at[:, :OUT].set(W4[:EMB])
    w4b = jnp.zeros((EMB, LANES), f32).at[:, :OUT].set(W4[EMB:])
    b4p = jnp.zeros((1, LANES), f32).at[0, :OUT].set(b4)

    pos3 = pos.reshape(B, NPTS, 3)
    post = jnp.swapaxes(pos3, 1, 2)

    rep = lambda shape: pl.BlockSpec(shape, lambda b: (0,) * len(shape))

    idx_pad, a_tab, c_tab = pl.pallas_call(
        _topk_kernel,
        grid=(B,),
        in_specs=[
            pl.BlockSpec((1, NPTS, 3), lambda b: (b, 0, 0)),
            pl.BlockSpec((1, 3, NPTS), lambda b: (b, 0, 0)),
            rep((3, 64)), rep((1, 64)), rep((3, LANES)),
        ],
        out_specs=[
            pl.BlockSpec((1, NPTS, LANES), lambda b: (b, 0, 0)),
            pl.BlockSpec((1, NPTS, 64), lambda b: (b, 0, 0)),
            pl.BlockSpec((1, NPTS, LANES), lambda b: (b, 0, 0)),
        ],
        out_shape=[
            jax.ShapeDtypeStruct((B, NPTS, LANES), jnp.int32),
            jax.ShapeDtypeStruct((B, NPTS, 64), f32),
            jax.ShapeDtypeStruct((B, NPTS, LANES), f32),
        ],
        scratch_shapes=[pltpu.VMEM((NPTS, NPTS), f32)],
    )(pos3, post, wa, ba, wc)

    idx_flat = jnp.transpose(idx_pad[:, :, :K], (2, 0, 1)).reshape(K * N)

    mesh = plsc.VectorSubcoreMesh(core_axis_name="c", subcore_axis_name="s")
    gathered = pl.kernel(
        _sc_gather,
        mesh=mesh,
        out_type=jax.ShapeDtypeStruct((K * N, LANES), f32),
        scratch_types=[
            pltpu.VMEM((_BPW,), jnp.int32),
            pltpu.VMEM((_SUPER, LANES), f32),
            pltpu.SemaphoreType.DMA,
        ],
    )(c_tab.reshape(N, LANES), idx_flat)
    neigh = gathered.reshape(K, B, NPTS, LANES)

    out = pl.pallas_call(
        _edge_kernel,
        grid=(B,),
        in_specs=[
            pl.BlockSpec((K, 1, NPTS, LANES), lambda b: (0, b, 0, 0)),
            pl.BlockSpec((1, NPTS, 64), lambda b: (b, 0, 0)),
            rep((64, 128)), rep((1, 128)),
            rep((128, EMB)), rep((1, EMB)),
            rep((EMB, LANES)), rep((EMB, LANES)), rep((1, LANES)),
        ],
        out_specs=pl.BlockSpec((1, 1, LANES), lambda b: (b, 0, 0)),
        out_shape=jax.ShapeDtypeStruct((B, 1, LANES), f32),
    )(neigh, a_tab, w2, b2e, w3, b3e, w4a, w4b, b4p)
    return out.reshape(B, LANES)[:, :OUT]


# speedup vs baseline: 1.2043x; 1.2043x over previous
import functools
import jax
import jax.numpy as jnp
from jax import lax
from jax.experimental import pallas as pl
from jax.experimental.pallas import tpu as pltpu
from jax.experimental.pallas import tpu_sc as plsc

B = 32
NPTS = 1024
N = B * NPTS
K = 20
EMB = 1024
OUT = 7
EPS = 1e-5
LANES = 128
BIG_F = 1e9

def _topk_kernel(pos_ref, post_ref, wa_ref, ba_ref, wc_ref, out_idx_ref,
                 a_ref, c_ref, dist_ref):
    b = pl.program_id(0)
    p = pos_ref[0]
    pt = post_ref[0]

    sq_r = jnp.sum(p * p, axis=1, keepdims=True)
    sq_c = jnp.sum(pt * pt, axis=0, keepdims=True)
    pp = jax.lax.dot(p, pt)
    dist_ref[...] = sq_r + sq_c - 2.0 * pp

    a_ref[0] = jax.lax.dot(p, wa_ref[...],
                           precision=jax.lax.Precision.HIGHEST) + ba_ref[...]
    c_ref[0] = jax.lax.dot(p, wc_ref[...],
                           precision=jax.lax.Precision.HIGHEST)

    iota = lax.broadcasted_iota(jnp.int32, (NPTS, NPTS), 1).astype(jnp.float32)
    lane = lax.broadcasted_iota(jnp.int32, (NPTS, LANES), 1)

    def body(k, acc):
        d = dist_ref[...]
        m = jnp.min(d, axis=1, keepdims=True)
        loc = jnp.where(d == m, iota, jnp.float32(BIG_F))
        idx = jnp.min(loc, axis=1, keepdims=True)
        oh = iota == idx
        dist_ref[...] = jnp.where(oh, jnp.inf, d)
        return jnp.where(lane == k, idx, acc)

    acc0 = jnp.zeros((NPTS, LANES), jnp.float32)
    acc = lax.fori_loop(0, K, body, acc0)
    out_idx_ref[0] = acc.astype(jnp.int32) + b * NPTS


_CHUNK = 128
_SUPER = 512
_NC = 2
_NS = 16
_NW = _NC * _NS


def _make_sc_gather(bpw):
    nsup = bpw // _SUPER

    def _sc_gather(table_hbm, idx_hbm, out_hbm, idx_v, stage_v, gsem):
        wid = lax.axis_index("s") * _NC + lax.axis_index("c")
        base = wid * bpw
        pltpu.sync_copy(idx_hbm.at[pl.ds(base, bpw)], idx_v)

        def body(g, carry):
            for c in range(_SUPER // _CHUNK):
                off = g * _SUPER + c * _CHUNK
                pltpu.async_copy(
                    table_hbm.at[idx_v.at[pl.ds(off, _CHUNK)]],
                    stage_v.at[pl.ds(c * _CHUNK, _CHUNK)], gsem)
            for c in range(_SUPER // _CHUNK):
                pltpu.make_async_copy(
                    table_hbm.at[
                        idx_v.at[pl.ds(g * _SUPER + c * _CHUNK, _CHUNK)]],
                    stage_v.at[pl.ds(c * _CHUNK, _CHUNK)], gsem).wait()
            pltpu.sync_copy(stage_v,
                            out_hbm.at[pl.ds(base + g * _SUPER, _SUPER)])
            return carry

        lax.fori_loop(0, nsup, body, 0)

    return _sc_gather


def _edge_kernel(neigh_ref, a_ref, w2_ref, b2_ref, w3_ref, b3_ref,
                 w4a_ref, w4b_ref, b4_ref, out_ref):
    a = a_ref[0]
    x1 = jnp.zeros((NPTS, 128), jnp.float32)
    for k in range(K):
        h1 = jnp.maximum(a + neigh_ref[k, 0, :, :64], 0.0)
        h2 = jnp.maximum(jax.lax.dot(h1, w2_ref[...]) + b2_ref[...], 0.0)
        x1 = jnp.maximum(x1, h2)
    x = jnp.maximum(jax.lax.dot(x1, w3_ref[...]) + b3_ref[...], 0.0)
    gmax = jnp.max(x, axis=0, keepdims=True)
    gmean = jnp.sum(x, axis=0, keepdims=True) * (1.0 / NPTS)
    o = (jax.lax.dot(gmax, w4a_ref[...]) + jax.lax.dot(gmean, w4b_ref[...])
         + b4_ref[...])
    out_ref[0] = o


def kernel(pos, batch, W1, b1, g1, be1, W2, b2, g2, be2, W3, b3, g3, be3,
           W4, b4):
    del batch
    f32 = jnp.float32
    s1 = g1 / jnp.sqrt(1.0 + EPS)
    s2 = g2 / jnp.sqrt(1.0 + EPS)
    s3 = g3 / jnp.sqrt(1.0 + EPS)

    wa = (W1[:3] - W1[3:]) * s1[None, :]
    ba = (b1 * s1 + be1).reshape(1, 64)
    wc = jnp.zeros((3, LANES), f32).at[:, :64].set(W1[3:] * s1[None, :])
    w2 = W2 * s2[None, :]
    b2e = (b2 * s2 + be2).reshape(1, 128)
    w3 = W3 * s3[None, :]
    b3e = (b3 * s3 + be3).reshape(1, EMB)
    w4a = jnp.zeros((EMB, LANES), f32).at[:, :OUT].set(W4[:EMB])
    w4b = jnp.zeros((EMB, LANES), f32).at[:, :OUT].set(W4[EMB:])
    b4p = jnp.zeros((1, LANES), f32).at[0, :OUT].set(b4)

    pos3 = pos.reshape(B, NPTS, 3)
    post = jnp.swapaxes(pos3, 1, 2)

    rep = lambda shape: pl.BlockSpec(shape, lambda b: (0,) * len(shape))

    nsplit = 2
    bh = B // nsplit
    nh = bh * NPTS
    mesh = plsc.VectorSubcoreMesh(core_axis_name="c", subcore_axis_name="s")
    outs = []
    for h in range(nsplit):
        p3 = lax.slice_in_dim(pos3, h * bh, (h + 1) * bh, axis=0)
        pt3 = lax.slice_in_dim(post, h * bh, (h + 1) * bh, axis=0)

        idx_pad, a_tab, c_tab = pl.pallas_call(
            _topk_kernel,
            grid=(bh,),
            in_specs=[
                pl.BlockSpec((1, NPTS, 3), lambda b: (b, 0, 0)),
                pl.BlockSpec((1, 3, NPTS), lambda b: (b, 0, 0)),
                rep((3, 64)), rep((1, 64)), rep((3, LANES)),
            ],
            out_specs=[
                pl.BlockSpec((1, NPTS, LANES), lambda b: (b, 0, 0)),
                pl.BlockSpec((1, NPTS, 64), lambda b: (b, 0, 0)),
                pl.BlockSpec((1, NPTS, LANES), lambda b: (b, 0, 0)),
            ],
            out_shape=[
                jax.ShapeDtypeStruct((bh, NPTS, LANES), jnp.int32),
                jax.ShapeDtypeStruct((bh, NPTS, 64), f32),
                jax.ShapeDtypeStruct((bh, NPTS, LANES), f32),
            ],
            scratch_shapes=[pltpu.VMEM((NPTS, NPTS), f32)],
        )(p3, pt3, wa, ba, wc)

        idx_flat = jnp.transpose(idx_pad[:, :, :K], (2, 0, 1)).reshape(K * nh)

        gathered = pl.kernel(
            _make_sc_gather((K * nh) // _NW),
            mesh=mesh,
            out_type=jax.ShapeDtypeStruct((K * nh, LANES), f32),
            scratch_types=[
                pltpu.VMEM(((K * nh) // _NW,), jnp.int32),
                pltpu.VMEM((_SUPER, LANES), f32),
                pltpu.SemaphoreType.DMA,
            ],
        )(c_tab.reshape(nh, LANES), idx_flat)
        neigh = gathered.reshape(K, bh, NPTS, LANES)

        out = pl.pallas_call(
            _edge_kernel,
            grid=(bh,),
            in_specs=[
                pl.BlockSpec((K, 1, NPTS, LANES), lambda b: (0, b, 0, 0)),
                pl.BlockSpec((1, NPTS, 64), lambda b: (b, 0, 0)),
                rep((64, 128)), rep((1, 128)),
                rep((128, EMB)), rep((1, EMB)),
                rep((EMB, LANES)), rep((EMB, LANES)), rep((1, LANES)),
            ],
            out_specs=pl.BlockSpec((1, 1, LANES), lambda b: (b, 0, 0)),
            out_shape=jax.ShapeDtypeStruct((bh, 1, LANES), f32),
        )(neigh, a_tab, w2, b2e, w3, b3e, w4a, w4b, b4p)
        outs.append(out.reshape(bh, LANES)[:, :OUT])
    return jnp.concatenate(outs, axis=0)


# speedup vs baseline: 1.2722x; 1.0564x over previous
import functools
import jax
import jax.numpy as jnp
from jax import lax
from jax.experimental import pallas as pl
from jax.experimental.pallas import tpu as pltpu
from jax.experimental.pallas import tpu_sc as plsc

B = 32
NPTS = 1024
N = B * NPTS
K = 20
EMB = 1024
OUT = 7
EPS = 1e-5
LANES = 128
BIG_F = 1e9

def _topk_kernel(pos_ref, post_ref, wa_ref, ba_ref, wc_ref, out_idx_ref,
                 a_ref, c_ref, dist_ref):
    b = pl.program_id(0)
    p = pos_ref[0]
    pt = post_ref[0]

    sq_r = jnp.sum(p * p, axis=1, keepdims=True)
    sq_c = jnp.sum(pt * pt, axis=0, keepdims=True)
    pp = jax.lax.dot(p, pt)
    dist_ref[...] = sq_r + sq_c - 2.0 * pp

    a_ref[0] = jax.lax.dot(p, wa_ref[...],
                           precision=jax.lax.Precision.HIGHEST) + ba_ref[...]
    c_ref[0] = jax.lax.dot(p, wc_ref[...],
                           precision=jax.lax.Precision.HIGHEST)

    iota = lax.broadcasted_iota(jnp.int32, (NPTS, NPTS), 1).astype(jnp.float32)
    lane = lax.broadcasted_iota(jnp.int32, (NPTS, LANES), 1)

    def body(k, acc):
        d = dist_ref[...]
        m = jnp.min(d, axis=1, keepdims=True)
        loc = jnp.where(d == m, iota, jnp.float32(BIG_F))
        idx = jnp.min(loc, axis=1, keepdims=True)
        oh = iota == idx
        dist_ref[...] = jnp.where(oh, jnp.inf, d)
        return jnp.where(lane == k, idx, acc)

    acc0 = jnp.zeros((NPTS, LANES), jnp.float32)
    acc = lax.fori_loop(0, K, body, acc0)
    out_idx_ref[0] = acc.astype(jnp.int32) + b * NPTS


_CHUNK = 128
_SUPER = 512
_NC = 2
_NS = 16
_NW = _NC * _NS


def _make_sc_gather(bpw):
    nsup = bpw // _SUPER

    def _sc_gather(table_hbm, idx_hbm, out_hbm, idx_v, stage_v, gsem):
        wid = lax.axis_index("s") * _NC + lax.axis_index("c")
        base = wid * bpw
        pltpu.sync_copy(idx_hbm.at[pl.ds(base, bpw)], idx_v)

        def body(g, carry):
            for c in range(_SUPER // _CHUNK):
                off = g * _SUPER + c * _CHUNK
                pltpu.async_copy(
                    table_hbm.at[idx_v.at[pl.ds(off, _CHUNK)]],
                    stage_v.at[pl.ds(c * _CHUNK, _CHUNK)], gsem)
            for c in range(_SUPER // _CHUNK):
                pltpu.make_async_copy(
                    table_hbm.at[
                        idx_v.at[pl.ds(g * _SUPER + c * _CHUNK, _CHUNK)]],
                    stage_v.at[pl.ds(c * _CHUNK, _CHUNK)], gsem).wait()
            pltpu.sync_copy(stage_v,
                            out_hbm.at[pl.ds(base + g * _SUPER, _SUPER)])
            return carry

        lax.fori_loop(0, nsup, body, 0)

    return _sc_gather


def _edge_kernel(neigh_ref, a_ref, w2_ref, b2_ref, w3_ref, b3_ref,
                 w4a_ref, w4b_ref, b4_ref, out_ref):
    a = a_ref[0]
    x1 = jnp.zeros((NPTS, 128), jnp.float32)
    for k in range(K):
        h1 = jnp.maximum(a + neigh_ref[k, 0, :, :64], 0.0)
        h2 = jnp.maximum(jax.lax.dot(h1, w2_ref[...]) + b2_ref[...], 0.0)
        x1 = jnp.maximum(x1, h2)
    x = jnp.maximum(jax.lax.dot(x1, w3_ref[...]) + b3_ref[...], 0.0)
    gmax = jnp.max(x, axis=0, keepdims=True)
    gmean = jnp.sum(x, axis=0, keepdims=True) * (1.0 / NPTS)
    o = (jax.lax.dot(gmax, w4a_ref[...]) + jax.lax.dot(gmean, w4b_ref[...])
         + b4_ref[...])
    out_ref[0] = o


def kernel(pos, batch, W1, b1, g1, be1, W2, b2, g2, be2, W3, b3, g3, be3,
           W4, b4):
    del batch
    f32 = jnp.float32
    s1 = g1 / jnp.sqrt(1.0 + EPS)
    s2 = g2 / jnp.sqrt(1.0 + EPS)
    s3 = g3 / jnp.sqrt(1.0 + EPS)

    wa = (W1[:3] - W1[3:]) * s1[None, :]
    ba = (b1 * s1 + be1).reshape(1, 64)
    wc = jnp.zeros((3, LANES), f32).at[:, :64].set(W1[3:] * s1[None, :])
    w2 = W2 * s2[None, :]
    b2e = (b2 * s2 + be2).reshape(1, 128)
    w3 = W3 * s3[None, :]
    b3e = (b3 * s3 + be3).reshape(1, EMB)
    w4a = jnp.zeros((EMB, LANES), f32).at[:, :OUT].set(W4[:EMB])
    w4b = jnp.zeros((EMB, LANES), f32).at[:, :OUT].set(W4[EMB:])
    b4p = jnp.zeros((1, LANES), f32).at[0, :OUT].set(b4)

    pos3 = pos.reshape(B, NPTS, 3)
    post = jnp.swapaxes(pos3, 1, 2)

    rep = lambda shape: pl.BlockSpec(shape, lambda b: (0,) * len(shape))

    nsplit = 4
    bh = B // nsplit
    nh = bh * NPTS
    mesh = plsc.VectorSubcoreMesh(core_axis_name="c", subcore_axis_name="s")
    outs = []
    for h in range(nsplit):
        p3 = lax.slice_in_dim(pos3, h * bh, (h + 1) * bh, axis=0)
        pt3 = lax.slice_in_dim(post, h * bh, (h + 1) * bh, axis=0)

        idx_pad, a_tab, c_tab = pl.pallas_call(
            _topk_kernel,
            grid=(bh,),
            in_specs=[
                pl.BlockSpec((1, NPTS, 3), lambda b: (b, 0, 0)),
                pl.BlockSpec((1, 3, NPTS), lambda b: (b, 0, 0)),
                rep((3, 64)), rep((1, 64)), rep((3, LANES)),
            ],
            out_specs=[
                pl.BlockSpec((1, NPTS, LANES), lambda b: (b, 0, 0)),
                pl.BlockSpec((1, NPTS, 64), lambda b: (b, 0, 0)),
                pl.BlockSpec((1, NPTS, LANES), lambda b: (b, 0, 0)),
            ],
            out_shape=[
                jax.ShapeDtypeStruct((bh, NPTS, LANES), jnp.int32),
                jax.ShapeDtypeStruct((bh, NPTS, 64), f32),
                jax.ShapeDtypeStruct((bh, NPTS, LANES), f32),
            ],
            scratch_shapes=[pltpu.VMEM((NPTS, NPTS), f32)],
        )(p3, pt3, wa, ba, wc)

        idx_flat = jnp.transpose(idx_pad[:, :, :K], (2, 0, 1)).reshape(K * nh)

        gathered = pl.kernel(
            _make_sc_gather((K * nh) // _NW),
            mesh=mesh,
            out_type=jax.ShapeDtypeStruct((K * nh, LANES), f32),
            scratch_types=[
                pltpu.VMEM(((K * nh) // _NW,), jnp.int32),
                pltpu.VMEM((_SUPER, LANES), f32),
                pltpu.SemaphoreType.DMA,
            ],
        )(c_tab.reshape(nh, LANES), idx_flat)
        neigh = gathered.reshape(K, bh, NPTS, LANES)

        out = pl.pallas_call(
            _edge_kernel,
            grid=(bh,),
            in_specs=[
                pl.BlockSpec((K, 1, NPTS, LANES), lambda b: (0, b, 0, 0)),
                pl.BlockSpec((1, NPTS, 64), lambda b: (b, 0, 0)),
                rep((64, 128)), rep((1, 128)),
                rep((128, EMB)), rep((1, EMB)),
                rep((EMB, LANES)), rep((EMB, LANES)), rep((1, LANES)),
            ],
            out_specs=pl.BlockSpec((1, 1, LANES), lambda b: (b, 0, 0)),
            out_shape=jax.ShapeDtypeStruct((bh, 1, LANES), f32),
        )(neigh, a_tab, w2, b2e, w3, b3e, w4a, w4b, b4p)
        outs.append(out.reshape(bh, LANES)[:, :OUT])
    return jnp.concatenate(outs, axis=0)
